# baseline (speedup 1.0000x reference)
import functools

import jax
import jax.numpy as jnp
from jax import lax
from jax.experimental import pallas as pl
from jax.experimental.pallas import tpu as pltpu
from jax.experimental.pallas import tpu_sc as plsc

B = 4096
HIST = 200
D = 128
FLAT = HIST * D
OUT = 512

CHUNKS = (512, 1536, 1536, 512)

NC = 2
NS = 16
NW = NC * NS
SPLIT = 104

BM = 256
HK = 40
NKT = HIST // HK


def _make_sc_gather(bc, base):
    spw = bc // NW

    def _sc_gather(idx_hbm, table_hbm, out_hbm, idx_v, rows_v, sem):
        wid = lax.axis_index("s") * NC + lax.axis_index("c")
        base_b = wid * spw

        def body(i, carry):
            b = base_b + i
            pltpu.sync_copy(idx_hbm.at[base + b], idx_v)
            cp1 = pltpu.async_copy(
                table_hbm.at[idx_v.at[pl.ds(0, SPLIT)]],
                rows_v.at[pl.ds(0, SPLIT)], sem)
            cp2 = pltpu.async_copy(
                table_hbm.at[idx_v.at[pl.ds(SPLIT, HIST - SPLIT)]],
                rows_v.at[pl.ds(SPLIT, HIST - SPLIT)], sem)
            cp1.wait()
            cp2.wait()
            pltpu.sync_copy(rows_v, out_hbm.at[b])
            return carry

        lax.fori_loop(0, spw, body, 0)

    return functools.partial(
        pl.kernel,
        out_type=jax.ShapeDtypeStruct((bc, HIST, D), jnp.float32),
        mesh=plsc.VectorSubcoreMesh(core_axis_name="c", subcore_axis_name="s"),
        scratch_types=[
            pltpu.VMEM((HIST,), jnp.int32),
            pltpu.VMEM((HIST, D), jnp.float32),
            pltpu.SemaphoreType.DMA,
        ],
        compiler_params=pltpu.CompilerParams(use_tc_tiling_on_sc=True),
    )(_sc_gather)


def _tc_body(x_ref, w1_ref, b1_ref, xa_ref, w2_ref, b2_ref, o_prev_ref,
             o_ref, acc_ref):
    k = pl.program_id(0)
    m = pl.program_id(1)
    del o_prev_ref
    part = None
    for j in range(HK // 2):
        xcat = jnp.concatenate(
            [x_ref[:, 2 * j, :], x_ref[:, 2 * j + 1, :]],
            axis=1).astype(jnp.bfloat16)
        d = jnp.dot(xcat, w1_ref[pl.ds(j * 2 * D, 2 * D), :],
                    preferred_element_type=jnp.float32)
        part = d if part is None else part + d

    @pl.when(k == 0)
    def _():
        acc_ref[m] = part

    @pl.when(k > 0)
    def _():
        acc_ref[m] += part

    @pl.when(k == NKT - 1)
    def _():
        o_ref[:, :OUT] = acc_ref[m] + b1_ref[...]
        o_ref[:, OUT:OUT + 2] = (
            jnp.dot(xa_ref[...], w2_ref[...],
                    preferred_element_type=jnp.float32) + b2_ref[...])


def _make_tc_matmul(bc, base):
    nmt = bc // BM
    base_m = base // BM

    return pl.pallas_call(
        _tc_body,
        grid=(NKT, nmt),
        in_specs=[
            pl.BlockSpec((BM, HK, D), lambda k, m: (m, k, 0)),
            pl.BlockSpec((HK * D, OUT), lambda k, m: (k, 0)),
            pl.BlockSpec((1, OUT), lambda k, m: (0, 0)),
            pl.BlockSpec((BM, 2), lambda k, m: (base_m + m, 0)),
            pl.BlockSpec((2, 2), lambda k, m: (0, 0)),
            pl.BlockSpec((1, 2), lambda k, m: (0, 0)),
            pl.BlockSpec((8, OUT + 2), lambda k, m: (0, 0)),
        ],
        out_specs=pl.BlockSpec((BM, OUT + 2), lambda k, m: (base_m + m, 0)),
        out_shape=jax.ShapeDtypeStruct((B, OUT + 2), jnp.float32),
        scratch_shapes=[pltpu.VMEM((nmt, BM, OUT), jnp.float32)],
        input_output_aliases={6: 0},
        compiler_params=pltpu.CompilerParams(
            dimension_semantics=("arbitrary", "arbitrary")),
    )


def kernel(x_rxdx, x_age_gender, table, W1, b1, W2, b2):
    W1bf = W1.astype(jnp.bfloat16)
    b1r = b1.reshape(1, OUT)
    b2r = b2.reshape(1, 2)
    out = jnp.zeros((B, OUT + 2), jnp.float32)
    base = 0
    for bc in CHUNKS:
        h1_c = _make_sc_gather(bc, base)(x_rxdx, table)
        out = _make_tc_matmul(bc, base)(
            h1_c, W1bf, b1r, x_age_gender, W2, b2r, out)
        base += bc
    return out


# speedup vs baseline: 1.0436x; 1.0436x over previous
import functools

import jax
import jax.numpy as jnp
from jax import lax
from jax.experimental import pallas as pl
from jax.experimental.pallas import tpu as pltpu
from jax.experimental.pallas import tpu_sc as plsc

B = 4096
HIST = 200
D = 128
FLAT = HIST * D
OUT = 512

CHUNKS = (1536, 1280, 768, 512)

NC = 2
NS = 16
NW = NC * NS
SPLIT = 104

BM = 256
HK = 40
NKT = HIST // HK


def _make_sc_gather(bc, base):
    spw = bc // NW

    def _sc_gather(idx_hbm, table_hbm, out_hbm, idx_v, rows_v, sem):
        wid = lax.axis_index("s") * NC + lax.axis_index("c")
        base_b = wid * spw

        def body(i, carry):
            b = base_b + i
            pltpu.sync_copy(idx_hbm.at[base + b], idx_v)
            cp1 = pltpu.async_copy(
                table_hbm.at[idx_v.at[pl.ds(0, SPLIT)]],
                rows_v.at[pl.ds(0, SPLIT)], sem)
            cp2 = pltpu.async_copy(
                table_hbm.at[idx_v.at[pl.ds(SPLIT, HIST - SPLIT)]],
                rows_v.at[pl.ds(SPLIT, HIST - SPLIT)], sem)
            cp1.wait()
            cp2.wait()
            pltpu.sync_copy(rows_v, out_hbm.at[b])
            return carry

        lax.fori_loop(0, spw, body, 0)

    return functools.partial(
        pl.kernel,
        out_type=jax.ShapeDtypeStruct((bc, HIST, D), jnp.float32),
        mesh=plsc.VectorSubcoreMesh(core_axis_name="c", subcore_axis_name="s"),
        scratch_types=[
            pltpu.VMEM((HIST,), jnp.int32),
            pltpu.VMEM((HIST, D), jnp.float32),
            pltpu.SemaphoreType.DMA,
        ],
        compiler_params=pltpu.CompilerParams(use_tc_tiling_on_sc=True),
    )(_sc_gather)


def _tc_body(x_ref, w1_ref, b1_ref, xa_ref, w2_ref, b2_ref, o_prev_ref,
             o_ref, acc_ref):
    k = pl.program_id(0)
    m = pl.program_id(1)
    del o_prev_ref
    part = None
    for j in range(HK // 2):
        xcat = jnp.concatenate(
            [x_ref[:, 2 * j, :], x_ref[:, 2 * j + 1, :]],
            axis=1).astype(jnp.bfloat16)
        d = jnp.dot(xcat, w1_ref[pl.ds(j * 2 * D, 2 * D), :],
                    preferred_element_type=jnp.float32)
        part = d if part is None else part + d

    @pl.when(k == 0)
    def _():
        acc_ref[m] = part

    @pl.when(k > 0)
    def _():
        acc_ref[m] += part

    @pl.when(k == NKT - 1)
    def _():
        o_ref[:, :OUT] = acc_ref[m] + b1_ref[...]
        o_ref[:, OUT:OUT + 2] = (
            jnp.dot(xa_ref[...], w2_ref[...],
                    preferred_element_type=jnp.float32) + b2_ref[...])


def _make_tc_matmul(bc, base):
    nmt = bc // BM
    base_m = base // BM

    return pl.pallas_call(
        _tc_body,
        grid=(NKT, nmt),
        in_specs=[
            pl.BlockSpec((BM, HK, D), lambda k, m: (m, k, 0)),
            pl.BlockSpec((HK * D, OUT), lambda k, m: (k, 0)),
            pl.BlockSpec((1, OUT), lambda k, m: (0, 0)),
            pl.BlockSpec((BM, 2), lambda k, m: (base_m + m, 0)),
            pl.BlockSpec((2, 2), lambda k, m: (0, 0)),
            pl.BlockSpec((1, 2), lambda k, m: (0, 0)),
            pl.BlockSpec((8, OUT + 2), lambda k, m: (0, 0)),
        ],
        out_specs=pl.BlockSpec((BM, OUT + 2), lambda k, m: (base_m + m, 0)),
        out_shape=jax.ShapeDtypeStruct((B, OUT + 2), jnp.float32),
        scratch_shapes=[pltpu.VMEM((nmt, BM, OUT), jnp.float32)],
        input_output_aliases={6: 0},
        compiler_params=pltpu.CompilerParams(
            dimension_semantics=("arbitrary", "arbitrary")),
    )


def kernel(x_rxdx, x_age_gender, table, W1, b1, W2, b2):
    W1bf = W1.astype(jnp.bfloat16)
    b1r = b1.reshape(1, OUT)
    b2r = b2.reshape(1, 2)
    out = jnp.zeros((B, OUT + 2), jnp.float32)
    base = 0
    for bc in CHUNKS:
        h1_c = _make_sc_gather(bc, base)(x_rxdx, table)
        out = _make_tc_matmul(bc, base)(
            h1_c, W1bf, b1r, x_age_gender, W2, b2r, out)
        base += bc
    return out


# speedup vs baseline: 1.0625x; 1.0181x over previous
import functools

import jax
import jax.numpy as jnp
from jax import lax
from jax.experimental import pallas as pl
from jax.experimental.pallas import tpu as pltpu
from jax.experimental.pallas import tpu_sc as plsc

B = 4096
HIST = 200
D = 128
FLAT = HIST * D
OUT = 512

CHUNKS = (1280, 1024, 768, 576, 448)

NC = 2
NS = 16
NW = NC * NS
SPLIT = 104

BM = 256
HK = 40
NKT = HIST // HK


def _make_sc_gather(bc, base):
    spw = bc // NW

    def _sc_gather(idx_hbm, table_hbm, out_hbm, idx_v, rows_v, sem):
        wid = lax.axis_index("s") * NC + lax.axis_index("c")
        base_b = wid * spw

        def body(i, carry):
            b = base_b + i
            pltpu.sync_copy(idx_hbm.at[base + b], idx_v)
            cp1 = pltpu.async_copy(
                table_hbm.at[idx_v.at[pl.ds(0, SPLIT)]],
                rows_v.at[pl.ds(0, SPLIT)], sem)
            cp2 = pltpu.async_copy(
                table_hbm.at[idx_v.at[pl.ds(SPLIT, HIST - SPLIT)]],
                rows_v.at[pl.ds(SPLIT, HIST - SPLIT)], sem)
            cp1.wait()
            cp2.wait()
            pltpu.sync_copy(rows_v, out_hbm.at[b])
            return carry

        lax.fori_loop(0, spw, body, 0)

    return functools.partial(
        pl.kernel,
        out_type=jax.ShapeDtypeStruct((bc, HIST, D), jnp.float32),
        mesh=plsc.VectorSubcoreMesh(core_axis_name="c", subcore_axis_name="s"),
        scratch_types=[
            pltpu.VMEM((HIST,), jnp.int32),
            pltpu.VMEM((HIST, D), jnp.float32),
            pltpu.SemaphoreType.DMA,
        ],
        compiler_params=pltpu.CompilerParams(use_tc_tiling_on_sc=True),
    )(_sc_gather)


def _tc_body(x_ref, w1_ref, b1_ref, xa_ref, w2_ref, b2_ref, o_prev_ref,
             o_ref, acc_ref):
    k = pl.program_id(0)
    m = pl.program_id(1)
    del o_prev_ref
    part = None
    for j in range(HK // 2):
        xcat = jnp.concatenate(
            [x_ref[:, 2 * j, :], x_ref[:, 2 * j + 1, :]],
            axis=1).astype(jnp.bfloat16)
        d = jnp.dot(xcat, w1_ref[pl.ds(j * 2 * D, 2 * D), :],
                    preferred_element_type=jnp.float32)
        part = d if part is None else part + d

    @pl.when(k == 0)
    def _():
        acc_ref[m] = part

    @pl.when(k > 0)
    def _():
        acc_ref[m] += part

    @pl.when(k == NKT - 1)
    def _():
        o_ref[:, :OUT] = acc_ref[m] + b1_ref[...]
        o_ref[:, OUT:OUT + 2] = (
            jnp.dot(xa_ref[...], w2_ref[...],
                    preferred_element_type=jnp.float32) + b2_ref[...])


def _make_tc_matmul(bc, base):
    nmt = bc // BM
    base_m = base // BM

    return pl.pallas_call(
        _tc_body,
        grid=(NKT, nmt),
        in_specs=[
            pl.BlockSpec((BM, HK, D), lambda k, m: (m, k, 0)),
            pl.BlockSpec((HK * D, OUT), lambda k, m: (k, 0)),
            pl.BlockSpec((1, OUT), lambda k, m: (0, 0)),
            pl.BlockSpec((BM, 2), lambda k, m: (base_m + m, 0)),
            pl.BlockSpec((2, 2), lambda k, m: (0, 0)),
            pl.BlockSpec((1, 2), lambda k, m: (0, 0)),
            pl.BlockSpec((8, OUT + 2), lambda k, m: (0, 0)),
        ],
        out_specs=pl.BlockSpec((BM, OUT + 2), lambda k, m: (base_m + m, 0)),
        out_shape=jax.ShapeDtypeStruct((B, OUT + 2), jnp.float32),
        scratch_shapes=[pltpu.VMEM((nmt, BM, OUT), jnp.float32)],
        input_output_aliases={6: 0},
        compiler_params=pltpu.CompilerParams(
            dimension_semantics=("arbitrary", "arbitrary")),
    )


def kernel(x_rxdx, x_age_gender, table, W1, b1, W2, b2):
    W1bf = W1.astype(jnp.bfloat16)
    b1r = b1.reshape(1, OUT)
    b2r = b2.reshape(1, 2)
    out = jnp.zeros((B, OUT + 2), jnp.float32)
    base = 0
    for bc in CHUNKS:
        h1_c = _make_sc_gather(bc, base)(x_rxdx, table)
        out = _make_tc_matmul(bc, base)(
            h1_c, W1bf, b1r, x_age_gender, W2, b2r, out)
        base += bc
    return out
